# baseline (speedup 1.0000x reference)
import functools

import jax
import jax.numpy as jnp
from jax import lax
from jax.experimental import pallas as pl
from jax.experimental.pallas import tpu as pltpu
from jax.experimental.pallas import tpu_sc as plsc

EMBED_DIM = 300
PAD_DIM = 304
BW = 16
NBAND = PAD_DIM // BW
NUM_CORES = 2
NUM_SUBCORES = 16
NUM_WORKERS = NUM_CORES * NUM_SUBCORES
NI, NJ = 4096, 200
IT = NI // 128
JT = NJ // 8
NCHUNK = NBAND * JT
LAST_W = EMBED_DIM - (NBAND - 1) * BW


def _make_gather():
  mesh = plsc.VectorSubcoreMesh(core_axis_name="c", subcore_axis_name="s")

  @functools.partial(
      pl.kernel,
      mesh=mesh,
      out_type=jax.ShapeDtypeStruct((EMBED_DIM, JT, IT, 8, 128), jnp.float32),
      scratch_types=[
          pltpu.VMEM((2, 8, 128), jnp.int32),
          pltpu.VMEM((2, 8, 128), jnp.int32),
          pltpu.VMEM((2, 1024, BW), jnp.float32),
          pltpu.VMEM((2, BW, 8, 128), jnp.float32),
          pltpu.SemaphoreType.DMA,
          pltpu.SemaphoreType.DMA,
          pltpu.SemaphoreType.DMA,
          pltpu.SemaphoreType.DMA,
          pltpu.SemaphoreType.DMA,
          pltpu.SemaphoreType.DMA,
      ],
      compiler_params=pltpu.CompilerParams(
          use_tc_tiling_on_sc=False, needs_layout_passes=False,
          disable_bounds_checks=True, disable_semaphore_checks=True),
  )
  def gather_kernel(xt_hbm, table_hbm, out_hbm,
                    idxr_v, idxs_v, gath_v, stg_v,
                    isem0, isem1, gsem0, gsem1, wsem0, wsem1):
    wid = lax.axis_index("s") * NUM_CORES + lax.axis_index("c")
    it = wid
    isems = (isem0, isem1)
    gsems = (gsem0, gsem1)
    wsems = (wsem0, wsem1)
    iota16 = lax.iota(jnp.int32, 16)


    def x_slice(c):
      jt = lax.rem(c, JT)
      return xt_hbm.at[pl.ds(jt * 8, 8), pl.ds(it * 128, 128)]

    def idx_stage(c, b):
      return pltpu.make_async_copy(x_slice(c), idxr_v.at[b], isems[b])

    def scale_idx(c, b):
      band = c // JT
      for s in range(8):
        for g in range(8):
          v = idxr_v[b, s, pl.ds(g * 16, 16)]
          idxs_v[b, s, pl.ds(g * 16, 16)] = v * NBAND + band

    def gathers(c, b):
      return [
          pltpu.make_async_copy(
              table_hbm.at[idxs_v.at[b, s]],
              gath_v.at[b, pl.ds(s * 128, 128)], gsems[b])
          for s in range(8)
      ]

    def transpose(b):
      def body(de, carry):
        col = jnp.full((16,), de, jnp.int32)
        for s in range(8):
          for g in range(8):
            vec = plsc.load_gather(
                gath_v.at[b], [s * 128 + g * 16 + iota16, col])
            stg_v[b, de, s, pl.ds(g * 16, 16)] = vec
        return carry
      lax.fori_loop(0, BW, body, 0)

    def do_write(c, b, action):
      band = c // JT
      jt = lax.rem(c, JT) if not isinstance(c, int) else c % JT
      tail = pltpu.make_async_copy(
          stg_v.at[b, pl.ds(0, LAST_W)],
          out_hbm.at[pl.ds((NBAND - 1) * BW, LAST_W), jt, it], wsems[b])
      if isinstance(c, int):
        if c // JT == NBAND - 1:
          getattr(tail, action)()
        else:
          full = pltpu.make_async_copy(
              stg_v.at[b], out_hbm.at[pl.ds((c // JT) * BW, BW), jt, it],
              wsems[b])
          getattr(full, action)()
        return
      full = pltpu.make_async_copy(
          stg_v.at[b],
          out_hbm.at[pl.ds(lax.min(band, NBAND - 2) * BW, BW), jt, it],
          wsems[b])

      @pl.when(band < NBAND - 1)
      def _():
        getattr(full, action)()

      @pl.when(band == NBAND - 1)
      def _():
        getattr(tail, action)()

    idx_stage(0, 0).start()
    idx_stage(0, 0).wait()
    scale_idx(0, 0)
    for cp in gathers(0, 0):
      cp.start()
    idx_stage(1, 1).start()

    def slot(c, b):
      @pl.when(c >= 2)
      def _():
        do_write(c - 2, b, "wait")
      for cp in gathers(c, b):
        cp.wait()
      transpose(b)
      do_write(c, b, "start")

      @pl.when(c <= NCHUNK - 2)
      def _():
        idx_stage(c + 1, 1 - b).wait()
        scale_idx(c + 1, 1 - b)
        for cp in gathers(c + 1, 1 - b):
          cp.start()

        @pl.when(c <= NCHUNK - 3)
        def _():
          idx_stage(c + 2, b).start()

    def loop_body(c2, carry):
      for u in (0, 1):
        c = c2 * 2 + u

        @pl.when(c < NCHUNK)
        def _():
          slot(c, u)
      return carry

    lax.fori_loop(0, (NCHUNK + 1) // 2, loop_body, 0)

    do_write(NCHUNK - 2, (NCHUNK - 2) % 2, "wait")
    do_write(NCHUNK - 1, (NCHUNK - 1) % 2, "wait")

  return gather_kernel


def kernel(x, table):
  table_p = jnp.pad(table, ((0, 0), (0, PAD_DIM - EMBED_DIM)))
  table_b = table_p.reshape(table.shape[0] * NBAND, BW)
  out5d = _make_gather()(x.T, table_b)
  return out5d.transpose(2, 4, 1, 3, 0).reshape(NI, NJ, EMBED_DIM)


# speedup vs baseline: 1.4765x; 1.4765x over previous
import functools

import jax
import jax.numpy as jnp
from jax import lax
from jax.experimental import pallas as pl
from jax.experimental.pallas import tpu as pltpu
from jax.experimental.pallas import tpu_sc as plsc

EMBED_DIM = 300
PAD_DIM = 304
HW = 152
NUM_CORES = 2
NUM_SUBCORES = 16
NI, NJ = 4096, 200
IT = NI // 128
JT = NJ // 8
LH = 64
NPAIR = NJ // 2
NGRP = NPAIR * 2
NCHK = NGRP * 4
H1_W = EMBED_DIM - HW


def _make_gather():
  mesh = plsc.VectorSubcoreMesh(core_axis_name="c", subcore_axis_name="s")

  @functools.partial(
      pl.kernel,
      mesh=mesh,
      out_type=jax.ShapeDtypeStruct((EMBED_DIM, JT, IT, 8, 128), jnp.float32),
      scratch_types=[
          pltpu.VMEM((4, LH), jnp.int32),
          pltpu.VMEM((2, LH), jnp.int32),
          pltpu.VMEM((2, LH, HW), jnp.float32),
          pltpu.VMEM((2, HW, 2, 128), jnp.float32),
          pltpu.SemaphoreType.DMA,
          pltpu.SemaphoreType.DMA,
          pltpu.SemaphoreType.DMA,
          pltpu.SemaphoreType.DMA,
          pltpu.SemaphoreType.DMA,
          pltpu.SemaphoreType.DMA,
      ],
      compiler_params=pltpu.CompilerParams(
          use_tc_tiling_on_sc=False, needs_layout_passes=False,
          disable_bounds_checks=True, disable_semaphore_checks=True),
  )
  def gather_kernel(xt_hbm, table_hbm, out_hbm,
                    idxr_v, idxs_v, rows_v, stg_v,
                    isem0, isem1, gsem0, gsem1, wsem0, wsem1):
    wid = lax.axis_index("s") * NUM_CORES + lax.axis_index("c")
    it = wid
    isems = (isem0, isem1)
    gsems = (gsem0, gsem1)
    wsems = (wsem0, wsem1)
    iota16 = lax.iota(jnp.int32, 16)


    def x_slice(p, h, c):
      jcol = 2 * p + (c // 2)
      return xt_hbm.at[jcol, pl.ds(it * 128 + (c % 2) * LH, LH)]

    def stage_idx(p, h, c, kmod4):
      return pltpu.make_async_copy(x_slice(p, h, c), idxr_v.at[kmod4],
                                   isems[kmod4 % 2])

    def scale_idx(h, kmod4, kmod2):
      for g in range(LH // 16):
        v = idxr_v[kmod4, pl.ds(g * 16, 16)]
        idxs_v[kmod2, pl.ds(g * 16, 16)] = v * 2 + h

    def gather(kmod2):
      return pltpu.make_async_copy(
          table_hbm.at[idxs_v.at[kmod2]], rows_v.at[kmod2], gsems[kmod2])

    def transpose(c, cbuf, sbuf):
      sj = c // 2
      l0 = (c % 2) * LH

      def body(e2, carry):
        for de in range(2):
          e = e2 * 2 + de
          col = jnp.full((16,), e, jnp.int32)
          for g in range(LH // 16):
            vec = plsc.load_gather(rows_v.at[cbuf],
                                   [g * 16 + iota16, col])
            stg_v[sbuf, e, sj, pl.ds(l0 + g * 16, 16)] = vec
        return carry

      lax.fori_loop(0, HW // 2, body, 0)

    def write(p, h, sbuf):
      jt = p // 4
      s0 = 2 * (p % 4) if isinstance(p, int) else 2 * lax.rem(p, 4)
      if h == 0:
        return pltpu.make_async_copy(
            stg_v.at[sbuf],
            out_hbm.at[pl.ds(0, HW), jt, it, pl.ds(s0, 2)], wsems[sbuf])
      return pltpu.make_async_copy(
          stg_v.at[sbuf, pl.ds(0, H1_W)],
          out_hbm.at[pl.ds(HW, H1_W), jt, it, pl.ds(s0, 2)], wsems[sbuf])

    def next_chunk(p, h, c):
      if c < 2:
        return p, h, c + 2
      if h == 0:
        return p, 1, c - 2
      return p + 1, 0, c - 2

    def next4_chunk(p, h, c):
      if h == 0:
        return p, 1, c
      return p + 1, 0, c

    for c in range(4):
      stage_idx(0, 0, c, c).start()
    for c in range(2):
      stage_idx(0, 0, c, c).wait()
      scale_idx(0, c, c)
      gather(c).start()

    def group_body(p, h, sbuf):
      @pl.when((p * 2 + h) >= 2)
      def _():
        write(p - 1, h, sbuf).wait()

      for c in range(4):
        kmod2 = c % 2
        gather(kmod2).wait()
        transpose(c, kmod2, sbuf)
        np_, nh, nc = next_chunk(p, h, c)

        @pl.when((p * 2 + h) * 4 + c + 2 <= NCHK - 1)
        def _():
          stage_idx(np_, nh, nc, (c + 2) % 4).wait()
          scale_idx(nh, (c + 2) % 4, kmod2)
          gather(kmod2).start()
          n4p, n4h, n4c = next4_chunk(p, h, c)

          @pl.when((p * 2 + h) * 4 + c + 4 <= NCHK - 1)
          def _():
            stage_idx(n4p, n4h, n4c, c).start()

      write(p, h, sbuf).start()

    def loop_body(p, carry):
      for h in (0, 1):
        sbuf = h
        group_body(p, h, sbuf)
      return carry

    lax.fori_loop(0, NPAIR, loop_body, 0)

    write(NPAIR - 1, 0, 0).wait()
    write(NPAIR - 1, 1, 1).wait()

  return gather_kernel


def kernel(x, table):
  table_p = jnp.pad(table, ((0, 0), (0, PAD_DIM - EMBED_DIM)))
  table_h = table_p.reshape(table.shape[0] * 2, HW)
  out5d = _make_gather()(x.T, table_h)
  return out5d.transpose(2, 4, 1, 3, 0).reshape(NI, NJ, EMBED_DIM)
